# Initial kernel scaffold; baseline (speedup 1.0000x reference)
import jax
import jax.numpy as jnp
from jax.experimental import pallas as pl


def kernel(x, edge_index, edge_weight, Ws, bs):
    raise NotImplementedError("write your pallas kernel here")


# speedup vs baseline: 31.4308x; 31.4308x over previous
import jax
import jax.numpy as jnp
from jax import lax
from jax.experimental import pallas as pl
from jax.experimental.pallas import tpu as pltpu
from jax.experimental.pallas import tpu_sc as plsc

N = 10000
NPAD = 10240
E = 320000
F = 16
NC = 2
NS = 16
NT = NC * NS
EPT_ALL = E // NS
EPT = E // NT
CH = 2000
HALF = NPAD // 2
DPT = HALF // NS
NPT = NPAD // NS

_mesh = plsc.VectorSubcoreMesh(core_axis_name="c", subcore_axis_name="s")


def _rsqrt_newton(x):
    i = lax.bitcast_convert_type(x, jnp.int32)
    i = 0x5F3759DF - lax.shift_right_arithmetic(i, 1)
    y = lax.bitcast_convert_type(i, jnp.float32)
    for _ in range(3):
        y = y * (1.5 - 0.5 * x * y * y)
    return y


def _deg_body(dst_hbm, ew_hbm, dinv_hbm, deg_sp, idx_v, val_v, buf_v):
    c = lax.axis_index("c")
    s = lax.axis_index("s")

    def init(j, _):
        buf_v[pl.ds(j * 16, 16)] = jnp.full((16,), 1.0, jnp.float32)
        return _

    lax.fori_loop(0, DPT // 16, init, None)
    pltpu.sync_copy(buf_v, deg_sp.at[pl.ds(s * DPT, DPT)])
    plsc.subcore_barrier()

    half_base = c * HALF
    for k in range(EPT_ALL // CH):
        eb = s * EPT_ALL + k * CH
        pltpu.sync_copy(dst_hbm.at[pl.ds(eb, CH)], idx_v)
        pltpu.sync_copy(ew_hbm.at[pl.ds(eb, CH)], val_v)

        def remap(j, _):
            v = idx_v[pl.ds(j * 16, 16)] - half_base
            ok = (v >= 0) & (v < HALF)
            idx_v[pl.ds(j * 16, 16)] = jnp.where(ok, v, HALF)
            return _

        lax.fori_loop(0, CH // 16, remap, None)
        pltpu.sync_copy(val_v, deg_sp.at[idx_v], add=True)
    plsc.subcore_barrier()

    pltpu.sync_copy(deg_sp.at[pl.ds(s * DPT, DPT)], buf_v)

    def rsq(j, _):
        buf_v[pl.ds(j * 16, 16)] = _rsqrt_newton(buf_v[pl.ds(j * 16, 16)])
        return _

    lax.fori_loop(0, DPT // 16, rsq, None)
    pltpu.sync_copy(buf_v, dinv_hbm.at[pl.ds(c * HALF + s * DPT, DPT)])


_deg_kernel = pl.kernel(
    _deg_body,
    out_type=jax.ShapeDtypeStruct((NPAD,), jnp.float32),
    mesh=_mesh,
    scratch_types=[
        pltpu.VMEM_SHARED((HALF + 16,), jnp.float32),
        pltpu.VMEM((CH,), jnp.int32),
        pltpu.VMEM((CH,), jnp.float32),
        pltpu.VMEM((DPT,), jnp.float32),
    ],
)


def _make_prop(compute_norm):
    def body(p_hbm, src_hbm, dst_hbm, ewn_hbm, dinv_hbm, acc_hbm, *rest):
        if compute_norm:
            norm_out, = rest[:1]
            rest = rest[1:]
        acc_sp, dinv_v, pbuf, sidx, didx, nrm, ewb, rows, sem = rest
        c = lax.axis_index("c")
        s = lax.axis_index("s")
        w = c * NS + s
        nb = s * NPT

        pltpu.sync_copy(dinv_hbm, dinv_v)
        pltpu.sync_copy(p_hbm.at[pl.ds(nb, NPT)], pbuf)
        cf = jnp.where(c == 0, 1.0, 0.0)

        def sinit(j, _):
            dv = dinv_v[pl.ds(nb + j * 16, 16)]
            dv2 = dv * dv * cf
            for l in range(16):
                n2 = j * 16 + l
                d = dv2[l]
                pbuf[n2, :] = pbuf[n2, :] * d
            return _

        lax.fori_loop(0, NPT // 16, sinit, None)
        pltpu.sync_copy(pbuf, acc_sp.at[pl.ds(nb, NPT)])
        plsc.subcore_barrier()

        for k in range(EPT // CH):
            eb = w * EPT + k * CH
            pltpu.sync_copy(src_hbm.at[pl.ds(eb, CH)], sidx)
            gather = pltpu.async_copy(p_hbm.at[sidx], rows, sem)
            pltpu.sync_copy(dst_hbm.at[pl.ds(eb, CH)], didx)
            if compute_norm:
                pltpu.sync_copy(ewn_hbm.at[pl.ds(eb, CH)], ewb)

                def mknorm(j, _):
                    sv = sidx[pl.ds(j * 16, 16)]
                    dv = didx[pl.ds(j * 16, 16)]
                    a = plsc.load_gather(dinv_v, [sv])
                    b2 = plsc.load_gather(dinv_v, [dv])
                    nrm[pl.ds(j * 16, 16)] = a * b2 * ewb[pl.ds(j * 16, 16)]
                    return _

                lax.fori_loop(0, CH // 16, mknorm, None)
                pltpu.sync_copy(nrm, norm_out.at[pl.ds(eb, CH)])
            else:
                pltpu.sync_copy(ewn_hbm.at[pl.ds(eb, CH)], nrm)
            gather.wait()

            def scale(j, _):
                nv = nrm[pl.ds(j * 16, 16)]
                for l in range(16):
                    e2 = j * 16 + l
                    rows[e2, :] = rows[e2, :] * nv[l]
                return _

            lax.fori_loop(0, CH // 16, scale, None)
            pltpu.sync_copy(rows, acc_sp.at[didx], add=True)
        plsc.subcore_barrier()
        pltpu.sync_copy(acc_sp.at[pl.ds(nb, NPT)], acc_hbm.at[c, pl.ds(nb, NPT)])

    out_type = [jax.ShapeDtypeStruct((NC, NPAD, F), jnp.float32)]
    if compute_norm:
        out_type.append(jax.ShapeDtypeStruct((E,), jnp.float32))
    return pl.kernel(
        body,
        out_type=tuple(out_type),
        mesh=_mesh,
        compiler_params=pltpu.CompilerParams(
            needs_layout_passes=False, use_tc_tiling_on_sc=False),
        scratch_types=[
            pltpu.VMEM_SHARED((NPAD, F), jnp.float32),
            pltpu.VMEM((NPAD,), jnp.float32),
            pltpu.VMEM((NPT, F), jnp.float32),
            pltpu.VMEM((CH,), jnp.int32),
            pltpu.VMEM((CH,), jnp.int32),
            pltpu.VMEM((CH,), jnp.float32),
            pltpu.VMEM((CH,), jnp.float32),
            pltpu.VMEM((CH, F), jnp.float32),
            pltpu.SemaphoreType.DMA,
        ],
    )


_prop_first = _make_prop(True)
_prop_rest = _make_prop(False)


def _mm_first_body(x_ref, w_ref, o_ref):
    o_ref[...] = jnp.dot(x_ref[...], w_ref[...],
                         preferred_element_type=jnp.float32)


_mm_first = pl.pallas_call(
    _mm_first_body,
    out_shape=jax.ShapeDtypeStruct((NPAD, F), jnp.float32),
)


def _mid_body(acc_ref, b_ref, w_ref, o_ref):
    h = jnp.maximum(acc_ref[0] + acc_ref[1] + b_ref[...], 0.0)
    o_ref[...] = jnp.dot(h, w_ref[...], preferred_element_type=jnp.float32)


_mm_mid = pl.pallas_call(
    _mid_body,
    out_shape=jax.ShapeDtypeStruct((NPAD, F), jnp.float32),
)


def _fin_body(acc_ref, b_ref, o_ref):
    o_ref[...] = jnp.maximum(acc_ref[0] + acc_ref[1] + b_ref[...], 0.0)


_mm_fin = pl.pallas_call(
    _fin_body,
    out_shape=jax.ShapeDtypeStruct((NPAD, F), jnp.float32),
)


def kernel(x, edge_index, edge_weight, Ws, bs):
    src = edge_index[0].astype(jnp.int32)
    dst = edge_index[1].astype(jnp.int32)
    ew = edge_weight.astype(jnp.float32)
    xp = jnp.pad(x, ((0, NPAD - N), (0, 0)))

    dinv = _deg_kernel(dst, ew)
    p = _mm_first(xp, Ws[0])
    acc, norm = _prop_first(p, src, dst, ew, dinv)
    for i in range(1, 13):
        p = _mm_mid(acc, bs[i - 1].reshape(1, F), Ws[i])
        (acc,) = _prop_rest(p, src, dst, norm, dinv)
    out = _mm_fin(acc, bs[12].reshape(1, F))
    return out[:N]


# speedup vs baseline: 35.4863x; 1.1290x over previous
import jax
import jax.numpy as jnp
from jax import lax
from jax.experimental import pallas as pl
from jax.experimental.pallas import tpu as pltpu
from jax.experimental.pallas import tpu_sc as plsc

N = 10000
NPAD = 10240
E = 320000
F = 16
NC = 2
NS = 16
NT = NC * NS
EPT_ALL = E // NS
EPT = E // NT
CH = 2000
HALF = NPAD // 2
DPT = HALF // NS
NPT = NPAD // NS

_mesh = plsc.VectorSubcoreMesh(core_axis_name="c", subcore_axis_name="s")


def _rsqrt_newton(x):
    i = lax.bitcast_convert_type(x, jnp.int32)
    i = 0x5F3759DF - lax.shift_right_arithmetic(i, 1)
    y = lax.bitcast_convert_type(i, jnp.float32)
    for _ in range(3):
        y = y * (1.5 - 0.5 * x * y * y)
    return y


def _deg_body(dst_hbm, ew_hbm, dinv_hbm, deg_sp, idx_v, val_v, buf_v):
    c = lax.axis_index("c")
    s = lax.axis_index("s")

    def init(j, _):
        buf_v[pl.ds(j * 16, 16)] = jnp.full((16,), 1.0, jnp.float32)
        return _

    lax.fori_loop(0, DPT // 16, init, None)
    pltpu.sync_copy(buf_v, deg_sp.at[pl.ds(s * DPT, DPT)])
    plsc.subcore_barrier()

    half_base = c * HALF
    for k in range(EPT_ALL // CH):
        eb = s * EPT_ALL + k * CH
        pltpu.sync_copy(dst_hbm.at[pl.ds(eb, CH)], idx_v)
        pltpu.sync_copy(ew_hbm.at[pl.ds(eb, CH)], val_v)

        def remap(j, _):
            v = idx_v[pl.ds(j * 16, 16)] - half_base
            ok = (v >= 0) & (v < HALF)
            idx_v[pl.ds(j * 16, 16)] = jnp.where(ok, v, HALF)
            return _

        lax.fori_loop(0, CH // 16, remap, None)
        pltpu.sync_copy(val_v, deg_sp.at[idx_v], add=True)
    plsc.subcore_barrier()

    pltpu.sync_copy(deg_sp.at[pl.ds(s * DPT, DPT)], buf_v)

    def rsq(j, _):
        buf_v[pl.ds(j * 16, 16)] = _rsqrt_newton(buf_v[pl.ds(j * 16, 16)])
        return _

    lax.fori_loop(0, DPT // 16, rsq, None)
    pltpu.sync_copy(buf_v, dinv_hbm.at[pl.ds(c * HALF + s * DPT, DPT)])


_deg_kernel = pl.kernel(
    _deg_body,
    out_type=jax.ShapeDtypeStruct((NPAD,), jnp.float32),
    mesh=_mesh,
    scratch_types=[
        pltpu.VMEM_SHARED((HALF + 16,), jnp.float32),
        pltpu.VMEM((CH,), jnp.int32),
        pltpu.VMEM((CH,), jnp.float32),
        pltpu.VMEM((DPT,), jnp.float32),
    ],
)


def _norm_body(src_hbm, dst_hbm, ew_hbm, dinv_hbm, norm_hbm,
               dinv_v, sidx, didx, ewb, nrm):
    c = lax.axis_index("c")
    s = lax.axis_index("s")
    w = c * NS + s
    pltpu.sync_copy(dinv_hbm, dinv_v)
    for k in range(EPT // CH):
        eb = w * EPT + k * CH
        pltpu.sync_copy(src_hbm.at[pl.ds(eb, CH)], sidx)
        pltpu.sync_copy(dst_hbm.at[pl.ds(eb, CH)], didx)
        pltpu.sync_copy(ew_hbm.at[pl.ds(eb, CH)], ewb)

        @plsc.parallel_loop(0, CH // 16, unroll=2)
        def mknorm(j):
            sv = sidx[pl.ds(j * 16, 16)]
            dv = didx[pl.ds(j * 16, 16)]
            a = plsc.load_gather(dinv_v, [sv])
            b2 = plsc.load_gather(dinv_v, [dv])
            nrm[pl.ds(j * 16, 16)] = a * b2 * ewb[pl.ds(j * 16, 16)]

        pltpu.sync_copy(nrm, norm_hbm.at[pl.ds(eb, CH)])


_norm_kernel = pl.kernel(
    _norm_body,
    out_type=jax.ShapeDtypeStruct((E,), jnp.float32),
    mesh=_mesh,
    compiler_params=pltpu.CompilerParams(
        needs_layout_passes=False, use_tc_tiling_on_sc=False),
    scratch_types=[
        pltpu.VMEM((NPAD,), jnp.float32),
        pltpu.VMEM((CH,), jnp.int32),
        pltpu.VMEM((CH,), jnp.int32),
        pltpu.VMEM((CH,), jnp.float32),
        pltpu.VMEM((CH,), jnp.float32),
    ],
)

NCH = EPT // CH


def _prop_body(p_hbm, src_hbm, dst_hbm, norm_hbm, dinv_hbm, acc_hbm,
               acc_sp, dinv_v, pbuf,
               sidx0, sidx1, didx0, didx1, nrm0, nrm1, rows0, rows1,
               gsem0, gsem1, ssem0, ssem1):
    sidx = (sidx0, sidx1)
    didx = (didx0, didx1)
    nrm = (nrm0, nrm1)
    rows = (rows0, rows1)
    gsem = (gsem0, gsem1)
    ssem = (ssem0, ssem1)
    c = lax.axis_index("c")
    s = lax.axis_index("s")
    w = c * NS + s
    nb = s * NPT
    eb0 = w * EPT

    pltpu.sync_copy(dinv_hbm.at[pl.ds(nb, NPT)], dinv_v)
    pltpu.sync_copy(p_hbm.at[pl.ds(nb, NPT)], pbuf)
    cf = jnp.where(c == 0, 1.0, 0.0)

    def sinit(j, _):
        dv = dinv_v[pl.ds(j * 16, 16)]
        dv2 = dv * dv * cf
        for l in range(16):
            n2 = j * 16 + l
            d = dv2[l]
            pbuf[n2, :] = pbuf[n2, :] * d
        return _

    lax.fori_loop(0, NPT // 16, sinit, None)
    pltpu.sync_copy(pbuf, acc_sp.at[pl.ds(nb, NPT)])
    plsc.subcore_barrier()

    def loads(k):
        b = k % 2
        eb = eb0 + k * CH
        pltpu.sync_copy(src_hbm.at[pl.ds(eb, CH)], sidx[b])
        g = pltpu.async_copy(p_hbm.at[sidx[b]], rows[b], gsem[b])
        pltpu.sync_copy(dst_hbm.at[pl.ds(eb, CH)], didx[b])
        pltpu.sync_copy(norm_hbm.at[pl.ds(eb, CH)], nrm[b])
        return g

    g = loads(0)
    scats = [None, None]
    for k in range(NCH):
        b = k % 2
        g.wait()
        if k + 1 < NCH:
            b2 = (k + 1) % 2
            if scats[b2] is not None:
                scats[b2].wait()
                scats[b2] = None
            g = loads(k + 1)

        @plsc.parallel_loop(0, CH // 16, unroll=2)
        def scale(j):
            nv = nrm[b][pl.ds(j * 16, 16)]
            for l in range(16):
                e2 = j * 16 + l
                rows[b][e2, :] = rows[b][e2, :] * nv[l]

        scats[b] = pltpu.async_copy(rows[b], acc_sp.at[didx[b]], ssem[b],
                                    add=True)
    for t in scats:
        if t is not None:
            t.wait()
    plsc.subcore_barrier()
    pltpu.sync_copy(acc_sp.at[pl.ds(nb, NPT)], acc_hbm.at[c, pl.ds(nb, NPT)])


_prop = pl.kernel(
    _prop_body,
    out_type=jax.ShapeDtypeStruct((NC, NPAD, F), jnp.float32),
    mesh=_mesh,
    compiler_params=pltpu.CompilerParams(
        needs_layout_passes=False, use_tc_tiling_on_sc=False),
    scratch_types=[
        pltpu.VMEM_SHARED((NPAD, F), jnp.float32),
        pltpu.VMEM((NPT,), jnp.float32),
        pltpu.VMEM((NPT, F), jnp.float32),
        pltpu.VMEM((CH,), jnp.int32),
        pltpu.VMEM((CH,), jnp.int32),
        pltpu.VMEM((CH,), jnp.int32),
        pltpu.VMEM((CH,), jnp.int32),
        pltpu.VMEM((CH,), jnp.float32),
        pltpu.VMEM((CH,), jnp.float32),
        pltpu.VMEM((CH, F), jnp.float32),
        pltpu.VMEM((CH, F), jnp.float32),
        pltpu.SemaphoreType.DMA,
        pltpu.SemaphoreType.DMA,
        pltpu.SemaphoreType.DMA,
        pltpu.SemaphoreType.DMA,
    ],
)


def _mm_first_body(x_ref, w_ref, o_ref):
    o_ref[...] = jnp.dot(x_ref[...], w_ref[...],
                         preferred_element_type=jnp.float32)


_mm_first = pl.pallas_call(
    _mm_first_body,
    out_shape=jax.ShapeDtypeStruct((NPAD, F), jnp.float32),
)


def _mid_body(acc_ref, b_ref, w_ref, o_ref):
    h = jnp.maximum(acc_ref[0] + acc_ref[1] + b_ref[...], 0.0)
    o_ref[...] = jnp.dot(h, w_ref[...], preferred_element_type=jnp.float32)


_mm_mid = pl.pallas_call(
    _mid_body,
    out_shape=jax.ShapeDtypeStruct((NPAD, F), jnp.float32),
)


def _fin_body(acc_ref, b_ref, o_ref):
    o_ref[...] = jnp.maximum(acc_ref[0] + acc_ref[1] + b_ref[...], 0.0)


_mm_fin = pl.pallas_call(
    _fin_body,
    out_shape=jax.ShapeDtypeStruct((NPAD, F), jnp.float32),
)


def kernel(x, edge_index, edge_weight, Ws, bs):
    src = edge_index[0].astype(jnp.int32)
    dst = edge_index[1].astype(jnp.int32)
    ew = edge_weight.astype(jnp.float32)
    xp = jnp.pad(x, ((0, NPAD - N), (0, 0)))

    dinv = _deg_kernel(dst, ew)
    norm = _norm_kernel(src, dst, ew, dinv)
    p = _mm_first(xp, Ws[0])
    acc = _prop(p, src, dst, norm, dinv)
    for i in range(1, 13):
        p = _mm_mid(acc, bs[i - 1].reshape(1, F), Ws[i])
        acc = _prop(p, src, dst, norm, dinv)
    out = _mm_fin(acc, bs[12].reshape(1, F))
    return out[:N]


# speedup vs baseline: 40.7663x; 1.1488x over previous
import jax
import jax.numpy as jnp
from jax import lax
from jax.experimental import pallas as pl
from jax.experimental.pallas import tpu as pltpu
from jax.experimental.pallas import tpu_sc as plsc

N = 10000
NPAD = 10240
E = 320000
F = 16
NC = 2
NS = 16
NT = NC * NS
EPT_ALL = E // NS
EPT = E // NT
CH = 2000
HALF = NPAD // 2
DPT = HALF // NS
NPT = NPAD // NS

_mesh = plsc.VectorSubcoreMesh(core_axis_name="c", subcore_axis_name="s")


def _rsqrt_newton(x):
    i = lax.bitcast_convert_type(x, jnp.int32)
    i = 0x5F3759DF - lax.shift_right_arithmetic(i, 1)
    y = lax.bitcast_convert_type(i, jnp.float32)
    for _ in range(3):
        y = y * (1.5 - 0.5 * x * y * y)
    return y


def _deg_body(dst_hbm, ew_hbm, dinv_hbm, deg_sp, idx_v, val_v, rows_v,
              dbuf_v, out_v):
    c = lax.axis_index("c")
    s = lax.axis_index("s")

    def init(j, _):
        dbuf_v[j, :] = jnp.full((16,), 1.0, jnp.float32)
        return _

    lax.fori_loop(0, DPT, init, None)
    pltpu.sync_copy(dbuf_v, deg_sp.at[pl.ds(s * DPT, DPT)])
    plsc.subcore_barrier()

    half_base = c * HALF
    for k in range(EPT_ALL // CH):
        eb = s * EPT_ALL + k * CH
        pltpu.sync_copy(dst_hbm.at[pl.ds(eb, CH)], idx_v)
        pltpu.sync_copy(ew_hbm.at[pl.ds(eb, CH)], val_v)

        @plsc.parallel_loop(0, CH // 16, unroll=2)
        def remap(j):
            v = idx_v[pl.ds(j * 16, 16)] - half_base
            ok = (v >= 0) & (v < HALF)
            idx_v[pl.ds(j * 16, 16)] = jnp.where(ok, v, HALF)
            ev = val_v[pl.ds(j * 16, 16)]
            for l in range(16):
                rows_v[j * 16 + l, :] = ev[l] * jnp.full((16,), 1.0,
                                                         jnp.float32)

        pltpu.sync_copy(rows_v, deg_sp.at[idx_v], add=True)
    plsc.subcore_barrier()

    pltpu.sync_copy(deg_sp.at[pl.ds(s * DPT, DPT)], dbuf_v)
    cz = jnp.zeros((16,), jnp.int32)
    ci = lax.iota(jnp.int32, 16)

    def rsq(j, _):
        dv = plsc.load_gather(dbuf_v, [j * 16 + ci, cz])
        out_v[pl.ds(j * 16, 16)] = _rsqrt_newton(dv)
        return _

    lax.fori_loop(0, DPT // 16, rsq, None)
    pltpu.sync_copy(out_v, dinv_hbm.at[pl.ds(c * HALF + s * DPT, DPT)])


_deg_kernel = pl.kernel(
    _deg_body,
    out_type=jax.ShapeDtypeStruct((NPAD,), jnp.float32),
    mesh=_mesh,
    compiler_params=pltpu.CompilerParams(
        needs_layout_passes=False, use_tc_tiling_on_sc=False),
    scratch_types=[
        pltpu.VMEM_SHARED((HALF + 16, 16), jnp.float32),
        pltpu.VMEM((CH,), jnp.int32),
        pltpu.VMEM((CH,), jnp.float32),
        pltpu.VMEM((CH, 16), jnp.float32),
        pltpu.VMEM((DPT, 16), jnp.float32),
        pltpu.VMEM((DPT,), jnp.float32),
    ],
)


def _norm_body(src_hbm, dst_hbm, ew_hbm, dinv_hbm, norm_hbm,
               dinv_v, sidx, didx, ewb, nrm):
    c = lax.axis_index("c")
    s = lax.axis_index("s")
    w = c * NS + s
    pltpu.sync_copy(dinv_hbm, dinv_v)
    for k in range(EPT // CH):
        eb = w * EPT + k * CH
        pltpu.sync_copy(src_hbm.at[pl.ds(eb, CH)], sidx)
        pltpu.sync_copy(dst_hbm.at[pl.ds(eb, CH)], didx)
        pltpu.sync_copy(ew_hbm.at[pl.ds(eb, CH)], ewb)

        @plsc.parallel_loop(0, CH // 16, unroll=2)
        def mknorm(j):
            sv = sidx[pl.ds(j * 16, 16)]
            dv = didx[pl.ds(j * 16, 16)]
            a = plsc.load_gather(dinv_v, [sv])
            b2 = plsc.load_gather(dinv_v, [dv])
            nrm[pl.ds(j * 16, 16)] = a * b2 * ewb[pl.ds(j * 16, 16)]

        pltpu.sync_copy(nrm, norm_hbm.at[pl.ds(eb, CH)])


_norm_kernel = pl.kernel(
    _norm_body,
    out_type=jax.ShapeDtypeStruct((E,), jnp.float32),
    mesh=_mesh,
    compiler_params=pltpu.CompilerParams(
        needs_layout_passes=False, use_tc_tiling_on_sc=False),
    scratch_types=[
        pltpu.VMEM((NPAD,), jnp.float32),
        pltpu.VMEM((CH,), jnp.int32),
        pltpu.VMEM((CH,), jnp.int32),
        pltpu.VMEM((CH,), jnp.float32),
        pltpu.VMEM((CH,), jnp.float32),
    ],
)

NCH = EPT // CH


def _make_prop(has_matmul):
    def body(*refs):
        if has_matmul:
            (accin_hbm, w_hbm, b_hbm, src_hbm, dst_hbm, norm_hbm, dinv_hbm,
             acc_hbm, acc_sp, p_sp, dinv_v, pbuf, a0buf, a1buf, wbuf, bbuf,
             sidx0, sidx1, didx0, didx1, nrm0, nrm1, rows0, rows1,
             gsem0, gsem1, ssem0, ssem1) = refs
        else:
            (p_hbm, src_hbm, dst_hbm, norm_hbm, dinv_hbm,
             acc_hbm, acc_sp, dinv_v, pbuf,
             sidx0, sidx1, didx0, didx1, nrm0, nrm1, rows0, rows1,
             gsem0, gsem1, ssem0, ssem1) = refs
        sidx = (sidx0, sidx1)
        didx = (didx0, didx1)
        nrm = (nrm0, nrm1)
        rows = (rows0, rows1)
        gsem = (gsem0, gsem1)
        ssem = (ssem0, ssem1)
        c = lax.axis_index("c")
        s = lax.axis_index("s")
        w = c * NS + s
        nb = s * NPT
        eb0 = w * EPT

        pltpu.sync_copy(dinv_hbm.at[pl.ds(nb, NPT)], dinv_v)
        if has_matmul:
            pltpu.sync_copy(w_hbm, wbuf)
            pltpu.sync_copy(b_hbm, bbuf)
            pltpu.sync_copy(accin_hbm.at[0, pl.ds(nb, NPT)], a0buf)
            pltpu.sync_copy(accin_hbm.at[1, pl.ds(nb, NPT)], a1buf)
            bv = bbuf[pl.ds(0, F)]
            wrows = [wbuf[k, :] for k in range(F)]

            @plsc.parallel_loop(0, NPT)
            def mm(j):
                v = jnp.maximum(a0buf[j, :] + a1buf[j, :] + bv, 0.0)
                acc16 = v[0] * wrows[0]
                for k2 in range(1, F):
                    acc16 = acc16 + v[k2] * wrows[k2]
                pbuf[j, :] = acc16

            pltpu.sync_copy(pbuf, p_sp.at[pl.ds(nb, NPT)])
        else:
            pltpu.sync_copy(p_hbm.at[pl.ds(nb, NPT)], pbuf)

        cf = jnp.where(c == 0, 1.0, 0.0)

        def sinit(j, _):
            dv = dinv_v[pl.ds(j * 16, 16)]
            dv2 = dv * dv * cf
            for l in range(16):
                n2 = j * 16 + l
                d = dv2[l]
                pbuf[n2, :] = pbuf[n2, :] * d
            return _

        lax.fori_loop(0, NPT // 16, sinit, None)
        pltpu.sync_copy(pbuf, acc_sp.at[pl.ds(nb, NPT)])
        plsc.subcore_barrier()

        gather_src = p_sp if has_matmul else p_hbm

        def loads(k):
            b = k % 2
            eb = eb0 + k * CH
            pltpu.sync_copy(src_hbm.at[pl.ds(eb, CH)], sidx[b])
            g = pltpu.async_copy(gather_src.at[sidx[b]], rows[b], gsem[b])
            pltpu.sync_copy(dst_hbm.at[pl.ds(eb, CH)], didx[b])
            pltpu.sync_copy(norm_hbm.at[pl.ds(eb, CH)], nrm[b])
            return g

        g = loads(0)
        scats = [None, None]
        for k in range(NCH):
            b = k % 2
            g.wait()
            if k + 1 < NCH:
                b2 = (k + 1) % 2
                if scats[b2] is not None:
                    scats[b2].wait()
                    scats[b2] = None
                g = loads(k + 1)

            @plsc.parallel_loop(0, CH // 16, unroll=2)
            def scale(j):
                nv = nrm[b][pl.ds(j * 16, 16)]
                for l in range(16):
                    e2 = j * 16 + l
                    rows[b][e2, :] = rows[b][e2, :] * nv[l]

            scats[b] = pltpu.async_copy(rows[b], acc_sp.at[didx[b]],
                                        ssem[b], add=True)
        for t in scats:
            if t is not None:
                t.wait()
        plsc.subcore_barrier()
        pltpu.sync_copy(acc_sp.at[pl.ds(nb, NPT)],
                        acc_hbm.at[c, pl.ds(nb, NPT)])

    scratch = [pltpu.VMEM_SHARED((NPAD, F), jnp.float32)]
    if has_matmul:
        scratch += [
            pltpu.VMEM_SHARED((NPAD, F), jnp.float32),
        ]
    scratch += [
        pltpu.VMEM((NPT,), jnp.float32),
        pltpu.VMEM((NPT, F), jnp.float32),
    ]
    if has_matmul:
        scratch += [
            pltpu.VMEM((NPT, F), jnp.float32),
            pltpu.VMEM((NPT, F), jnp.float32),
            pltpu.VMEM((F, F), jnp.float32),
            pltpu.VMEM((F,), jnp.float32),
        ]
    scratch += [
        pltpu.VMEM((CH,), jnp.int32),
        pltpu.VMEM((CH,), jnp.int32),
        pltpu.VMEM((CH,), jnp.int32),
        pltpu.VMEM((CH,), jnp.int32),
        pltpu.VMEM((CH,), jnp.float32),
        pltpu.VMEM((CH,), jnp.float32),
        pltpu.VMEM((CH, F), jnp.float32),
        pltpu.VMEM((CH, F), jnp.float32),
        pltpu.SemaphoreType.DMA,
        pltpu.SemaphoreType.DMA,
        pltpu.SemaphoreType.DMA,
        pltpu.SemaphoreType.DMA,
    ]
    return pl.kernel(
        body,
        out_type=jax.ShapeDtypeStruct((NC, NPAD, F), jnp.float32),
        mesh=_mesh,
        compiler_params=pltpu.CompilerParams(
            needs_layout_passes=False, use_tc_tiling_on_sc=False),
        scratch_types=scratch,
    )


_prop = _make_prop(False)
_prop_mm = _make_prop(True)


def _mm_first_body(x_ref, w_ref, o_ref):
    o_ref[...] = jnp.dot(x_ref[...], w_ref[...],
                         preferred_element_type=jnp.float32)


_mm_first = pl.pallas_call(
    _mm_first_body,
    out_shape=jax.ShapeDtypeStruct((NPAD, F), jnp.float32),
)


def _mid_body(acc_ref, b_ref, w_ref, o_ref):
    h = jnp.maximum(acc_ref[0] + acc_ref[1] + b_ref[...], 0.0)
    o_ref[...] = jnp.dot(h, w_ref[...], preferred_element_type=jnp.float32)


_mm_mid = pl.pallas_call(
    _mid_body,
    out_shape=jax.ShapeDtypeStruct((NPAD, F), jnp.float32),
)


def _fin_body(acc_ref, b_ref, o_ref):
    o_ref[...] = jnp.maximum(acc_ref[0] + acc_ref[1] + b_ref[...], 0.0)


_mm_fin = pl.pallas_call(
    _fin_body,
    out_shape=jax.ShapeDtypeStruct((NPAD, F), jnp.float32),
)


def kernel(x, edge_index, edge_weight, Ws, bs):
    src = edge_index[0].astype(jnp.int32)
    dst = edge_index[1].astype(jnp.int32)
    ew = edge_weight.astype(jnp.float32)
    xp = jnp.pad(x, ((0, NPAD - N), (0, 0)))

    dinv = _deg_kernel(dst, ew)
    norm = _norm_kernel(src, dst, ew, dinv)
    p = _mm_first(xp, Ws[0])
    acc = _prop(p, src, dst, norm, dinv)
    for i in range(1, 13):
        acc = _prop_mm(acc, Ws[i], bs[i - 1], src, dst, norm, dinv)
    out = _mm_fin(acc, bs[12].reshape(1, F))
    return out[:N]


# speedup vs baseline: 55.8307x; 1.3695x over previous
import jax
import jax.numpy as jnp
from jax import lax
from jax.experimental import pallas as pl
from jax.experimental.pallas import tpu as pltpu
from jax.experimental.pallas import tpu_sc as plsc

N = 10000
NPAD = 10240
E = 320000
F = 16
NC = 2
NS = 16
NT = NC * NS
EPT_ALL = E // NS
EPT = E // NT
CH = 2000
HALF = NPAD // 2
DPT = HALF // NS
NPT = NPAD // NS

_mesh = plsc.VectorSubcoreMesh(core_axis_name="c", subcore_axis_name="s")


def _rsqrt_newton(x):
    i = lax.bitcast_convert_type(x, jnp.int32)
    i = 0x5F3759DF - lax.shift_right_arithmetic(i, 1)
    y = lax.bitcast_convert_type(i, jnp.float32)
    for _ in range(3):
        y = y * (1.5 - 0.5 * x * y * y)
    return y


def _deg_body(dst_hbm, ew_hbm, dinv_hbm, deg_sp, idx_v, val_v, rows_v,
              dbuf_v, out_v):
    c = lax.axis_index("c")
    s = lax.axis_index("s")

    def init(j, _):
        dbuf_v[j, :] = jnp.full((16,), 1.0, jnp.float32)
        return _

    lax.fori_loop(0, DPT, init, None)
    pltpu.sync_copy(dbuf_v, deg_sp.at[pl.ds(s * DPT, DPT)])
    plsc.subcore_barrier()

    half_base = c * HALF
    for k in range(EPT_ALL // CH):
        eb = s * EPT_ALL + k * CH
        pltpu.sync_copy(dst_hbm.at[pl.ds(eb, CH)], idx_v)
        pltpu.sync_copy(ew_hbm.at[pl.ds(eb, CH)], val_v)

        @plsc.parallel_loop(0, CH // 16, unroll=2)
        def remap(j):
            v = idx_v[pl.ds(j * 16, 16)] - half_base
            ok = (v >= 0) & (v < HALF)
            idx_v[pl.ds(j * 16, 16)] = jnp.where(ok, v, v & 4095)
            ev = jnp.where(ok, val_v[pl.ds(j * 16, 16)], 0.0)
            for l in range(16):
                rows_v[j * 16 + l, :] = ev[l] * jnp.full((16,), 1.0,
                                                         jnp.float32)

        pltpu.sync_copy(rows_v, deg_sp.at[idx_v], add=True)
    plsc.subcore_barrier()

    pltpu.sync_copy(deg_sp.at[pl.ds(s * DPT, DPT)], dbuf_v)
    cz = jnp.zeros((16,), jnp.int32)
    ci = lax.iota(jnp.int32, 16)

    def rsq(j, _):
        dv = plsc.load_gather(dbuf_v, [j * 16 + ci, cz])
        out_v[pl.ds(j * 16, 16)] = _rsqrt_newton(dv)
        return _

    lax.fori_loop(0, DPT // 16, rsq, None)
    pltpu.sync_copy(out_v, dinv_hbm.at[pl.ds(c * HALF + s * DPT, DPT)])


_deg_kernel = pl.kernel(
    _deg_body,
    out_type=jax.ShapeDtypeStruct((NPAD,), jnp.float32),
    mesh=_mesh,
    compiler_params=pltpu.CompilerParams(
        needs_layout_passes=False, use_tc_tiling_on_sc=False),
    scratch_types=[
        pltpu.VMEM_SHARED((HALF + 16, 16), jnp.float32),
        pltpu.VMEM((CH,), jnp.int32),
        pltpu.VMEM((CH,), jnp.float32),
        pltpu.VMEM((CH, 16), jnp.float32),
        pltpu.VMEM((DPT, 16), jnp.float32),
        pltpu.VMEM((DPT,), jnp.float32),
    ],
)


def _norm_body(src_hbm, dst_hbm, ew_hbm, dinv_hbm, norm_hbm,
               dinv_v, sidx, didx, ewb, nrm):
    c = lax.axis_index("c")
    s = lax.axis_index("s")
    w = c * NS + s
    pltpu.sync_copy(dinv_hbm, dinv_v)
    for k in range(EPT // CH):
        eb = w * EPT + k * CH
        pltpu.sync_copy(src_hbm.at[pl.ds(eb, CH)], sidx)
        pltpu.sync_copy(dst_hbm.at[pl.ds(eb, CH)], didx)
        pltpu.sync_copy(ew_hbm.at[pl.ds(eb, CH)], ewb)

        @plsc.parallel_loop(0, CH // 16, unroll=2)
        def mknorm(j):
            sv = sidx[pl.ds(j * 16, 16)]
            dv = didx[pl.ds(j * 16, 16)]
            a = plsc.load_gather(dinv_v, [sv])
            b2 = plsc.load_gather(dinv_v, [dv])
            nrm[pl.ds(j * 16, 16)] = a * b2 * ewb[pl.ds(j * 16, 16)]

        pltpu.sync_copy(nrm, norm_hbm.at[pl.ds(eb, CH)])


_norm_kernel = pl.kernel(
    _norm_body,
    out_type=jax.ShapeDtypeStruct((E,), jnp.float32),
    mesh=_mesh,
    compiler_params=pltpu.CompilerParams(
        needs_layout_passes=False, use_tc_tiling_on_sc=False),
    scratch_types=[
        pltpu.VMEM((NPAD,), jnp.float32),
        pltpu.VMEM((CH,), jnp.int32),
        pltpu.VMEM((CH,), jnp.int32),
        pltpu.VMEM((CH,), jnp.float32),
        pltpu.VMEM((CH,), jnp.float32),
    ],
)

NCH = EPT // CH


CHP = 1000
NCHP = EPT // CHP


def _make_prop(has_matmul):
    def body(*refs):
        if has_matmul:
            (accin_hbm, w_hbm, b_hbm, src_hbm, dst_hbm, norm_hbm, dinv_hbm,
             acc_hbm, acc_sp, p_sp, dinv_v, pbuf, a0buf, a1buf, wbuf, bbuf,
             src_f, dst_f, nrm_f, rows0, rows1,
             lsem, gsem0, gsem1, ssem0, ssem1) = refs
        else:
            (p_hbm, src_hbm, dst_hbm, norm_hbm, dinv_hbm,
             acc_hbm, acc_sp, dinv_v, pbuf,
             src_f, dst_f, nrm_f, rows0, rows1,
             lsem, gsem0, gsem1, ssem0, ssem1) = refs
        rows = (rows0, rows1)
        gsem = (gsem0, gsem1)
        ssem = (ssem0, ssem1)
        c = lax.axis_index("c")
        s = lax.axis_index("s")
        w = c * NS + s
        nb = s * NPT
        eb0 = w * EPT

        lds = [pltpu.async_copy(dinv_hbm.at[pl.ds(nb, NPT)], dinv_v, lsem)]
        for k in range(NCHP):
            eb = eb0 + k * CHP
            lds.append(pltpu.async_copy(src_hbm.at[pl.ds(eb, CHP)],
                                        src_f.at[k], lsem))
            lds.append(pltpu.async_copy(dst_hbm.at[pl.ds(eb, CHP)],
                                        dst_f.at[k], lsem))
            lds.append(pltpu.async_copy(norm_hbm.at[pl.ds(eb, CHP)],
                                        nrm_f.at[k], lsem))
        if has_matmul:
            lds.append(pltpu.async_copy(w_hbm, wbuf, lsem))
            lds.append(pltpu.async_copy(b_hbm, bbuf, lsem))
            lds.append(pltpu.async_copy(accin_hbm.at[0, pl.ds(nb, NPT)],
                                        a0buf, lsem))
            lds.append(pltpu.async_copy(accin_hbm.at[1, pl.ds(nb, NPT)],
                                        a1buf, lsem))
        else:
            lds.append(pltpu.async_copy(p_hbm.at[pl.ds(nb, NPT)], pbuf,
                                        lsem))
        for d in lds:
            d.wait()

        if has_matmul:
            bv = bbuf[pl.ds(0, F)]
            wrows = [wbuf[k, :] for k in range(F)]

            @plsc.parallel_loop(0, NPT)
            def mm(j):
                v = jnp.maximum(a0buf[j, :] + a1buf[j, :] + bv, 0.0)
                acc16 = v[0] * wrows[0]
                for k2 in range(1, F):
                    acc16 = acc16 + v[k2] * wrows[k2]
                pbuf[j, :] = acc16

            pltpu.sync_copy(pbuf, p_sp.at[pl.ds(nb, NPT)])

        cf = jnp.where(c == 0, 1.0, 0.0)

        def sinit(j, _):
            dv = dinv_v[pl.ds(j * 16, 16)]
            dv2 = dv * dv * cf
            for l in range(16):
                n2 = j * 16 + l
                d = dv2[l]
                pbuf[n2, :] = pbuf[n2, :] * d
            return _

        lax.fori_loop(0, NPT // 16, sinit, None)
        pltpu.sync_copy(pbuf, acc_sp.at[pl.ds(nb, NPT)])
        plsc.subcore_barrier()

        gather_src = p_sp if has_matmul else p_hbm

        def gather_start(k):
            return pltpu.async_copy(gather_src.at[src_f.at[k]],
                                    rows[k % 2], gsem[k % 2])

        g = gather_start(0)
        scats = [None, None]
        for k in range(NCHP):
            b = k % 2
            g.wait()
            if k + 1 < NCHP:
                b2 = (k + 1) % 2
                if scats[b2] is not None:
                    scats[b2].wait()
                    scats[b2] = None
                g = gather_start(k + 1)

            @plsc.parallel_loop(0, CHP // 16, unroll=2)
            def scale(j):
                nv = nrm_f[k, pl.ds(j * 16, 16)]
                for l in range(16):
                    e2 = j * 16 + l
                    rows[b][e2, :] = rows[b][e2, :] * nv[l]

            scats[b] = pltpu.async_copy(rows[b], acc_sp.at[dst_f.at[k]],
                                        ssem[b], add=True)
        for t in scats:
            if t is not None:
                t.wait()
        plsc.subcore_barrier()
        pltpu.sync_copy(acc_sp.at[pl.ds(nb, NPT)],
                        acc_hbm.at[c, pl.ds(nb, NPT)])

    scratch = [pltpu.VMEM_SHARED((NPAD, F), jnp.float32)]
    if has_matmul:
        scratch += [
            pltpu.VMEM_SHARED((NPAD, F), jnp.float32),
        ]
    scratch += [
        pltpu.VMEM((NPT,), jnp.float32),
        pltpu.VMEM((NPT, F), jnp.float32),
    ]
    if has_matmul:
        scratch += [
            pltpu.VMEM((NPT, F), jnp.float32),
            pltpu.VMEM((NPT, F), jnp.float32),
            pltpu.VMEM((F, F), jnp.float32),
            pltpu.VMEM((F,), jnp.float32),
        ]
    scratch += [
        pltpu.VMEM((NCHP, CHP), jnp.int32),
        pltpu.VMEM((NCHP, CHP), jnp.int32),
        pltpu.VMEM((NCHP, CHP), jnp.float32),
        pltpu.VMEM((CHP, F), jnp.float32),
        pltpu.VMEM((CHP, F), jnp.float32),
        pltpu.SemaphoreType.DMA,
        pltpu.SemaphoreType.DMA,
        pltpu.SemaphoreType.DMA,
        pltpu.SemaphoreType.DMA,
        pltpu.SemaphoreType.DMA,
    ]
    return pl.kernel(
        body,
        out_type=jax.ShapeDtypeStruct((NC, NPAD, F), jnp.float32),
        mesh=_mesh,
        compiler_params=pltpu.CompilerParams(
            needs_layout_passes=False, use_tc_tiling_on_sc=False),
        scratch_types=scratch,
    )


_prop = _make_prop(False)
_prop_mm = _make_prop(True)


def _mm_first_body(x_ref, w_ref, o_ref):
    o_ref[...] = jnp.dot(x_ref[...], w_ref[...],
                         preferred_element_type=jnp.float32)


_mm_first = pl.pallas_call(
    _mm_first_body,
    out_shape=jax.ShapeDtypeStruct((NPAD, F), jnp.float32),
)


def _mid_body(acc_ref, b_ref, w_ref, o_ref):
    h = jnp.maximum(acc_ref[0] + acc_ref[1] + b_ref[...], 0.0)
    o_ref[...] = jnp.dot(h, w_ref[...], preferred_element_type=jnp.float32)


_mm_mid = pl.pallas_call(
    _mid_body,
    out_shape=jax.ShapeDtypeStruct((NPAD, F), jnp.float32),
)


def _fin_body(acc_ref, b_ref, o_ref):
    o_ref[...] = jnp.maximum(acc_ref[0] + acc_ref[1] + b_ref[...], 0.0)


_mm_fin = pl.pallas_call(
    _fin_body,
    out_shape=jax.ShapeDtypeStruct((NPAD, F), jnp.float32),
)


def kernel(x, edge_index, edge_weight, Ws, bs):
    src = edge_index[0].astype(jnp.int32)
    dst = edge_index[1].astype(jnp.int32)
    ew = edge_weight.astype(jnp.float32)
    xp = jnp.pad(x, ((0, NPAD - N), (0, 0)))

    dinv = _deg_kernel(dst, ew)
    norm = _norm_kernel(src, dst, ew, dinv)
    p = _mm_first(xp, Ws[0])
    acc = _prop(p, src, dst, norm, dinv)
    for i in range(1, 13):
        acc = _prop_mm(acc, Ws[i], bs[i - 1], src, dst, norm, dinv)
    out = _mm_fin(acc, bs[12].reshape(1, F))
    return out[:N]
